# baseline (speedup 1.0000x reference)
import functools

import jax
import jax.numpy as jnp
from jax import lax
from jax.experimental import pallas as pl
from jax.experimental.pallas import tpu as pltpu
from jax.experimental.pallas import tpu_sc as plsc

N = 10000
NE = 320000
F = 128
H = 16
G = 256
NC, NS, L = 2, 16, 16
NW = NC * NS
NEP = 327680
ROWS2D = NEP // 128
EPT = NEP // NW
CH = 1024
GRP = CH // 128
CPW = EPT // CH
RGPW = EPT // 128
NPAD = 10240
HALF = NPAD // 2
HALF1 = HALF + 1
ACCW = H * HALF1

_mesh = plsc.VectorSubcoreMesh(core_axis_name="c", subcore_axis_name="s")
_sc_params = pltpu.CompilerParams(needs_layout_passes=False,
                                  use_tc_tiling_on_sc=False)


@functools.partial(
    pl.kernel,
    out_type=jax.ShapeDtypeStruct((NW * NPAD,), jnp.float32),
    mesh=_mesh,
    compiler_params=_sc_params,
    scratch_types=[
        pltpu.VMEM((NPAD,), jnp.float32),
        pltpu.VMEM((GRP, 128), jnp.int32),
        pltpu.VMEM((GRP, 128), jnp.float32),
    ],
)
def _sc_deg(dst_hbm, w_hbm, out_hbm, acc, dstb, wb):
    cid = lax.axis_index("c")
    sid = lax.axis_index("s")
    wid = sid * NC + cid
    zero16 = jnp.zeros((L,), jnp.float32)
    ji = lax.iota(jnp.int32, L)

    def _zero(i, c):
        acc[pl.ds(i * L, L)] = zero16
        return c
    lax.fori_loop(0, NPAD // L, _zero, 0)

    def _chunk(c, carry):
        row0 = wid * RGPW + c * GRP
        pltpu.sync_copy(dst_hbm.at[pl.ds(row0, GRP)], dstb)
        pltpu.sync_copy(w_hbm.at[pl.ds(row0, GRP)], wb)

        def _vreg(gk, c2):
            g = gk // (128 // L)
            j = gk % (128 // L)
            dv = dstb[g, pl.ds(j * L, L)]
            wv = wb[g, pl.ds(j * L, L)]
            for lane in range(L):
                plsc.addupdate_scatter(acc, [dv], wv, mask=ji == lane)
            return c2
        lax.fori_loop(0, CH // L, _vreg, 0)
        return carry
    lax.fori_loop(0, CPW, _chunk, 0)
    pltpu.sync_copy(acc, out_hbm.at[pl.ds(wid * NPAD, NPAD)])


def _make_sc_layer(compute_norm):
    n_out = ([jax.ShapeDtypeStruct((ROWS2D, 128), jnp.float32)] if compute_norm else []) + [
        jax.ShapeDtypeStruct((NW * ACCW,), jnp.float32),
        jax.ShapeDtypeStruct((NW * ACCW,), jnp.float32),
    ]
    scratch = [
        pltpu.VMEM((GRP, 128), jnp.int32),
        pltpu.VMEM((GRP, 128), jnp.int32),
        pltpu.VMEM((GRP, 128), jnp.float32),
        pltpu.VMEM((GRP, 128, H), jnp.float32),
        pltpu.VMEM((ACCW,), jnp.float32),
        pltpu.SemaphoreType.DMA,
    ]
    if compute_norm:
        scratch = [pltpu.VMEM((GRP, 128), jnp.float32),
                   pltpu.VMEM((N,), jnp.float32),
                   ] + scratch

    def body(*refs):
        if compute_norm:
            (src_hbm, dst_hbm, w_hbm, dinv_hbm, h_hbm,
             norm_hbm, plo_hbm, phi_hbm,
             wb, dinv_v, srcb, dstb, normb, rows, acc, sem) = refs
        else:
            (src_hbm, dst_hbm, norm_hbm, h_hbm,
             plo_hbm, phi_hbm,
             srcb, dstb, normb, rows, acc, sem) = refs
        cid = lax.axis_index("c")
        sid = lax.axis_index("s")
        wid = sid * NC + cid
        ji = lax.iota(jnp.int32, L)
        jv = ji * HALF1
        zero16 = jnp.zeros((L,), jnp.float32)
        if compute_norm:
            pltpu.sync_copy(dinv_hbm, dinv_v)

        for p in range(2):
            lo = p * HALF
            out_hbm = plo_hbm if p == 0 else phi_hbm

            def _zero(i, c):
                acc[pl.ds(i * L, L)] = zero16
                return c
            lax.fori_loop(0, ACCW // L, _zero, 0)

            def _chunk(c, carry):
                row0 = wid * RGPW + c * GRP
                pltpu.sync_copy(src_hbm.at[pl.ds(row0, GRP)], srcb)
                pltpu.sync_copy(dst_hbm.at[pl.ds(row0, GRP)], dstb)
                if compute_norm:
                    pltpu.sync_copy(w_hbm.at[pl.ds(row0, GRP)], wb)
                else:
                    pltpu.sync_copy(norm_hbm.at[pl.ds(row0, GRP)], normb)
                descs = [pltpu.async_copy(h_hbm.at[srcb.at[g]], rows.at[g], sem)
                         for g in range(GRP)]
                for d in descs:
                    d.wait()

                def _vreg(gk, c2):
                    g = gk // (128 // L)
                    j = gk % (128 // L)
                    dv = dstb[g, pl.ds(j * L, L)]
                    if compute_norm:
                        sv = srcb[g, pl.ds(j * L, L)]
                        wv = wb[g, pl.ds(j * L, L)]
                        nv = (plsc.load_gather(dinv_v, [sv]) * wv
                              * plsc.load_gather(dinv_v, [dv]))
                        if p == 0:
                            normb[g, pl.ds(j * L, L)] = nv
                    else:
                        nv = normb[g, pl.ds(j * L, L)]
                    nz = jnp.minimum((dv - lo).astype(jnp.uint32),
                                     jnp.uint32(HALF)).astype(jnp.int32)
                    for lane in range(L):
                        e = j * L + lane
                        plsc.addupdate_scatter(acc, [nz[lane] + jv],
                                               rows[g, e, :] * nv[lane])
                    return c2
                lax.fori_loop(0, CH // L, _vreg, 0)
                if compute_norm and p == 0:
                    pltpu.sync_copy(normb, norm_hbm.at[pl.ds(row0, GRP)])
                return carry
            lax.fori_loop(0, CPW, _chunk, 0)
            pltpu.sync_copy(acc, out_hbm.at[pl.ds(wid * ACCW, ACCW)])

    return functools.partial(pl.kernel, out_type=n_out, mesh=_mesh,
                             compiler_params=_sc_params,
                             scratch_types=scratch)(body)


_sc_layer0 = _make_sc_layer(True)
_sc_layer1 = _make_sc_layer(False)


def _bn_rows(s, g, b):
    m = jnp.mean(s, axis=0, keepdims=True)
    v = jnp.mean((s - m) * (s - m), axis=0, keepdims=True)
    return g * (s - m) * lax.rsqrt(v + 1e-5) + b


def _bn_cols(s, g, b):
    m = jnp.mean(s, axis=1, keepdims=True)
    v = jnp.mean((s - m) * (s - m), axis=1, keepdims=True)
    return g * (s - m) * lax.rsqrt(v + 1e-5) + b


def _reduce_halves(q0, q1):
    s0 = jnp.sum(q0[...], axis=0)[:, :HALF]
    s1 = jnp.sum(q1[...], axis=0)[:, :HALF]
    return jnp.concatenate([s0, s1], axis=1)[:, :N]


def _tc1_body(x, wemb, bemb, gemb, beemb, wc0, degp,
              h0lin_o, h0lint_o, dinv_o, dinvsq_o):
    s = jnp.dot(x[...], wemb[...], preferred_element_type=jnp.float32) + bemb[...]
    h = jax.nn.relu(_bn_rows(s, gemb[...], beemb[...]))
    h0lin_o[...] = jnp.dot(h, wc0[...], preferred_element_type=jnp.float32)
    h0lint_o[...] = lax.dot_general(wc0[...], h, (((0,), (1,)), ((), ())),
                                    preferred_element_type=jnp.float32)[:, :N]
    deg = jnp.sum(degp[...], axis=0) + 1.0
    dinv_o[...] = lax.rsqrt(deg)[:N]
    dinvsq_o[...] = (1.0 / deg)[:N].reshape(1, N)


def _tc1(x, wemb, bemb, gemb, beemb, wc0, degp):
    return pl.pallas_call(
        _tc1_body,
        out_shape=[
            jax.ShapeDtypeStruct((N, H), jnp.float32),
            jax.ShapeDtypeStruct((H, N), jnp.float32),
            jax.ShapeDtypeStruct((N,), jnp.float32),
            jax.ShapeDtypeStruct((1, N), jnp.float32),
        ],
    )(x, wemb, bemb, gemb, beemb, wc0, degp)


def _tc2_body(q0, q1, h0lint, dinvsq, bc0, g0, be0, wc1, h1lin_o, h1lint_o):
    aggt = _reduce_halves(q0, q1) + dinvsq[...] * h0lint[...] + bc0[...]
    h0t = jax.nn.relu(_bn_cols(aggt, g0[...], be0[...]))
    h1lin_o[...] = lax.dot_general(h0t, wc1[...], (((0,), (0,)), ((), ())),
                                   preferred_element_type=jnp.float32)
    h1lint_o[...] = lax.dot_general(wc1[...], h0t, (((0,), (0,)), ((), ())),
                                    preferred_element_type=jnp.float32)


def _tc2(q0, q1, h0lint, dinvsq, bc0, g0, be0, wc1):
    return pl.pallas_call(
        _tc2_body,
        out_shape=[
            jax.ShapeDtypeStruct((N, H), jnp.float32),
            jax.ShapeDtypeStruct((H, N), jnp.float32),
        ],
    )(q0, q1, h0lint, dinvsq, bc0, g0, be0, wc1)


def _tc3_body(q0, q1, h1lint, dinvsq, bc1, g1, be1, bids,
              speed, wsp, bsp, gsp, besp,
              ra, rb, wrc, brc, grc, berc, wrl, brl,
              wo0, bo0, go, beo, wo1, bo1, out_o):
    aggt = _reduce_halves(q0, q1) + dinvsq[...] * h1lint[...] + bc1[...]
    h1t = jax.nn.relu(_bn_cols(aggt, g1[...], be1[...]))
    oht = (lax.broadcasted_iota(jnp.int32, (G, N), 0) == bids[...]).astype(jnp.float32)
    psum = lax.dot_general(oht, h1t, (((1,), (1,)), ((), ())),
                           preferred_element_type=jnp.float32)
    counts = jnp.sum(oht, axis=1, keepdims=True)
    pooled = psum / jnp.maximum(counts, 1.0)
    sv = jnp.dot(speed[...], wsp[...], preferred_element_type=jnp.float32) + bsp[...]
    venc = jax.nn.relu(_bn_rows(sv, gsp[...], besp[...]))
    w = wrc[...]
    zpad = jnp.zeros((G, 1), jnp.float32)
    rap = jnp.concatenate([zpad, ra[...], zpad], axis=1)
    rbp = jnp.concatenate([zpad, rb[...], zpad], axis=1)
    rc = brc[...]
    for k in range(3):
        rc = rc + rap[:, k:k + 10] * w[0:1, k:k + 1] + rbp[:, k:k + 10] * w[1:2, k:k + 1]
    m = jnp.mean(rc)
    v = jnp.mean((rc - m) * (rc - m))
    rcn = jax.nn.relu(grc[...] * (rc - m) * lax.rsqrt(v + 1e-5) + berc[...])
    renc = jnp.dot(rcn, wrl[...], preferred_element_type=jnp.float32) + brl[...]
    hh = jnp.concatenate([pooled, venc, renc], axis=1)
    s0 = jnp.dot(hh, wo0[...], preferred_element_type=jnp.float32) + bo0[...]
    a = jax.nn.relu(_bn_rows(s0, go[...], beo[...]))
    out_o[...] = jnp.dot(a, wo1[...], preferred_element_type=jnp.float32) + bo1[...]


def _tc3(*args):
    return pl.pallas_call(
        _tc3_body,
        out_shape=jax.ShapeDtypeStruct((G, 8), jnp.float32),
    )(*args)


def kernel(x, edge_weight, speed, route, params, edge_index, batch_ids):
    p = params
    src = edge_index[0]
    dst = edge_index[1]
    pad = NEP - NE
    zi = jnp.zeros((pad,), jnp.int32)
    srcp = jnp.concatenate([src, zi]).reshape(ROWS2D, 128)
    dstp = jnp.concatenate([dst, zi]).reshape(ROWS2D, 128)
    wp = jnp.concatenate([edge_weight, jnp.zeros((pad,), jnp.float32)]).reshape(ROWS2D, 128)

    degp = _sc_deg(dstp, wp).reshape(NW, NPAD)
    h0lin, h0lint, dinv, dinvsq = _tc1(
        x, p['W_emb'], p['b_emb'], p['g_emb'], p['be_emb'], p['W_c0'], degp)
    bn_shape = (H, 1)
    norm2d, q00, q01 = _sc_layer0(srcp, dstp, wp, dinv, h0lin)
    h1lin, h1lint = _tc2(q00.reshape(NW, H, HALF1), q01.reshape(NW, H, HALF1),
                         h0lint, dinvsq,
                         p['b_c0'].reshape(bn_shape), p['g_bn0'].reshape(bn_shape),
                         p['be_bn0'].reshape(bn_shape), p['W_c1'])
    q10, q11 = _sc_layer1(srcp, dstp, norm2d, h1lin)
    out = _tc3(q10.reshape(NW, H, HALF1), q11.reshape(NW, H, HALF1),
               h1lint, dinvsq,
               p['b_c1'].reshape(bn_shape), p['g_bn1'].reshape(bn_shape),
               p['be_bn1'].reshape(bn_shape),
               batch_ids.reshape(1, N),
               speed, p['W_sp'], p['b_sp'], p['g_sp'], p['be_sp'],
               route[:, :, 0], route[:, :, 1],
               p['W_rc'].reshape(2, 3), p['b_rc'], p['g_rc'], p['be_rc'],
               p['W_rl'], p['b_rl'],
               p['W_o0'], p['b_o0'], p['g_o'], p['be_o'], p['W_o1'], p['b_o1'])
    return out


# speedup vs baseline: 3.0646x; 3.0646x over previous
import functools

import jax
import jax.numpy as jnp
from jax import lax
from jax.experimental import pallas as pl
from jax.experimental.pallas import tpu as pltpu
from jax.experimental.pallas import tpu_sc as plsc

N = 10000
NE = 320000
F = 128
H = 16
G = 256
NC, NS, L = 2, 16, 16
NW = NC * NS
NEP = 327680
ROWS2D = NEP // 128
EPT = NEP // NW
CH = 2048
GRP = CH // 128
CPW = EPT // CH
RGPW = EPT // 128
NPAD = 10240
NPT = NPAD // NS

_mesh = plsc.VectorSubcoreMesh(core_axis_name="c", subcore_axis_name="s")
_sc_params = pltpu.CompilerParams(needs_layout_passes=False,
                                  use_tc_tiling_on_sc=False)


@functools.partial(
    pl.kernel,
    out_type=jax.ShapeDtypeStruct((NW * NPAD,), jnp.float32),
    mesh=_mesh,
    compiler_params=_sc_params,
    scratch_types=[
        pltpu.VMEM((NPAD,), jnp.float32),
        pltpu.VMEM((GRP, 128), jnp.int32),
        pltpu.VMEM((GRP, 128), jnp.float32),
    ],
)
def _sc_deg(dst_hbm, w_hbm, out_hbm, acc, dstb, wb):
    cid = lax.axis_index("c")
    sid = lax.axis_index("s")
    wid = sid * NC + cid
    zero16 = jnp.zeros((L,), jnp.float32)
    ji = lax.iota(jnp.int32, L)

    def _zero(i, c):
        acc[pl.ds(i * L, L)] = zero16
        return c
    lax.fori_loop(0, NPAD // L, _zero, 0)

    def _chunk(c, carry):
        row0 = wid * RGPW + c * GRP
        pltpu.sync_copy(dst_hbm.at[pl.ds(row0, GRP)], dstb)
        pltpu.sync_copy(w_hbm.at[pl.ds(row0, GRP)], wb)

        def _vreg(gk, c2):
            g = gk // (128 // L)
            j = gk % (128 // L)
            dv = dstb[g, pl.ds(j * L, L)]
            wv = wb[g, pl.ds(j * L, L)]
            for lane in range(L):
                plsc.addupdate_scatter(acc, [dv], wv, mask=ji == lane)
            return c2
        lax.fori_loop(0, CH // L, _vreg, 0)
        return carry
    lax.fori_loop(0, CPW, _chunk, 0)
    pltpu.sync_copy(acc, out_hbm.at[pl.ds(wid * NPAD, NPAD)])


def _make_sc_layer(compute_norm):
    n_out = ([jax.ShapeDtypeStruct((ROWS2D, 128), jnp.float32)] if compute_norm else []) + [
        jax.ShapeDtypeStruct((NPAD, H), jnp.float32),
        jax.ShapeDtypeStruct((NPAD, H), jnp.float32),
    ]
    scratch = [
        pltpu.VMEM((2, GRP, 128), jnp.int32),
        pltpu.VMEM((2, GRP, 128), jnp.int32),
        pltpu.VMEM((2, GRP, 128), jnp.float32),
        pltpu.VMEM((2, GRP, 128, H), jnp.float32),
        pltpu.VMEM((NPT, H), jnp.float32),
        pltpu.VMEM_SHARED((NPAD, H), jnp.float32),
        pltpu.SemaphoreType.DMA,
        pltpu.SemaphoreType.DMA,
    ]
    if compute_norm:
        scratch = [pltpu.VMEM((2, GRP, 128), jnp.float32),
                   pltpu.VMEM((N,), jnp.float32),
                   ] + scratch

    def body(*refs):
        if compute_norm:
            (src_hbm, dst_hbm, w_hbm, dinv_hbm, h_hbm,
             norm_hbm, p0_hbm, p1_hbm,
             wb, dinv_v, srcb, dstb, normb, rows, zbuf, spacc, sem0, sem1) = refs
        else:
            (src_hbm, dst_hbm, norm_hbm, h_hbm,
             p0_hbm, p1_hbm,
             srcb, dstb, normb, rows, zbuf, spacc, sem0, sem1) = refs
        sems = (sem0, sem1)
        cid = lax.axis_index("c")
        sid = lax.axis_index("s")
        wid = sid * NC + cid
        zero16 = jnp.zeros((H,), jnp.float32)

        def _zrow(i, c):
            zbuf[i, :] = zero16
            return c
        lax.fori_loop(0, NPT, _zrow, 0)
        pltpu.sync_copy(zbuf, spacc.at[pl.ds(sid * NPT, NPT)])
        if compute_norm:
            pltpu.sync_copy(dinv_hbm, dinv_v)
        plsc.subcore_barrier()

        def _load_idx(c, b):
            row0 = wid * RGPW + c * GRP
            pltpu.sync_copy(src_hbm.at[pl.ds(row0, GRP)], srcb.at[b])
            pltpu.sync_copy(dst_hbm.at[pl.ds(row0, GRP)], dstb.at[b])
            if compute_norm:
                pltpu.sync_copy(w_hbm.at[pl.ds(row0, GRP)], wb.at[b])
            else:
                pltpu.sync_copy(norm_hbm.at[pl.ds(row0, GRP)], normb.at[b])

        def _fire_gather(b):
            return [pltpu.async_copy(h_hbm.at[srcb.at[b].at[g]],
                                     rows.at[b].at[g], sems[b])
                    for g in range(GRP)]

        _load_idx(0, 0)
        pend = _fire_gather(0)
        for c in range(CPW):
            b = c % 2
            nb = (c + 1) % 2
            if c + 1 < CPW:
                _load_idx(c + 1, nb)
                nxt = _fire_gather(nb)
            for d in pend:
                d.wait()
            if c + 1 < CPW:
                pend = nxt

            def _vreg(gk, c2):
                g = gk // (128 // L)
                j = gk % (128 // L)
                if compute_norm:
                    sv = srcb[b, g, pl.ds(j * L, L)]
                    dv = dstb[b, g, pl.ds(j * L, L)]
                    wv = wb[b, g, pl.ds(j * L, L)]
                    nv = (plsc.load_gather(dinv_v, [sv]) * wv
                          * plsc.load_gather(dinv_v, [dv]))
                    normb[b, g, pl.ds(j * L, L)] = nv
                else:
                    nv = normb[b, g, pl.ds(j * L, L)]
                for lane in range(L):
                    e = j * L + lane
                    rows[b, g, e, :] = rows[b, g, e, :] * nv[lane]
                return c2
            lax.fori_loop(0, CH // L, _vreg, 0)
            for g in range(GRP):
                pltpu.sync_copy(rows.at[b].at[g], spacc.at[dstb.at[b].at[g]],
                                add=True)
            if compute_norm:
                row0 = wid * RGPW + c * GRP
                pltpu.sync_copy(normb.at[b], norm_hbm.at[pl.ds(row0, GRP)])
        plsc.subcore_barrier()

        @pl.when(cid == 0)
        def _():
            pltpu.sync_copy(spacc.at[pl.ds(sid * NPT, NPT)],
                            p0_hbm.at[pl.ds(sid * NPT, NPT)])

        @pl.when(cid == 1)
        def _():
            pltpu.sync_copy(spacc.at[pl.ds(sid * NPT, NPT)],
                            p1_hbm.at[pl.ds(sid * NPT, NPT)])

    return functools.partial(pl.kernel, out_type=n_out, mesh=_mesh,
                             compiler_params=_sc_params,
                             scratch_types=scratch)(body)


_sc_layer0 = _make_sc_layer(True)
_sc_layer1 = _make_sc_layer(False)


def _bn_rows(s, g, b):
    m = jnp.mean(s, axis=0, keepdims=True)
    v = jnp.mean((s - m) * (s - m), axis=0, keepdims=True)
    return g * (s - m) * lax.rsqrt(v + 1e-5) + b


def _tc1_body(x, wemb, bemb, gemb, beemb, wc0, degp,
              h0lin_o, dinv_o, dinvsq_o):
    s = jnp.dot(x[...], wemb[...], preferred_element_type=jnp.float32) + bemb[...]
    h = jax.nn.relu(_bn_rows(s, gemb[...], beemb[...]))
    h0lin_o[...] = jnp.dot(h, wc0[...], preferred_element_type=jnp.float32)
    deg = jnp.sum(degp[...], axis=0)[:N] + 1.0
    dinv_o[...] = lax.rsqrt(deg)
    dinvsq_o[...] = 1.0 / deg


def _tc1(x, wemb, bemb, gemb, beemb, wc0, degp):
    return pl.pallas_call(
        _tc1_body,
        out_shape=[
            jax.ShapeDtypeStruct((N, H), jnp.float32),
            jax.ShapeDtypeStruct((N,), jnp.float32),
            jax.ShapeDtypeStruct((N,), jnp.float32),
        ],
    )(x, wemb, bemb, gemb, beemb, wc0, degp)


def _tc2_body(p0, p1, h0lin, dinvsq2, bc0, g0, be0, wc1, h1lin_o):
    agg = p0[...] + p1[...] + dinvsq2[...] * h0lin[...] + bc0[...]
    h0 = jax.nn.relu(_bn_rows(agg, g0[...], be0[...]))
    h1lin_o[...] = jnp.dot(h0, wc1[...], preferred_element_type=jnp.float32)


def _tc2(p0, p1, h0lin, dinvsq2, bc0, g0, be0, wc1):
    return pl.pallas_call(
        _tc2_body,
        out_shape=jax.ShapeDtypeStruct((N, H), jnp.float32),
    )(p0, p1, h0lin, dinvsq2, bc0, g0, be0, wc1)


def _tc3_body(p0, p1, h1lin, dinvsq2, bc1, g1, be1, bids,
              speed, wsp, bsp, gsp, besp,
              ra, rb, wrc, brc, grc, berc, wrl, brl,
              wo0, bo0, go, beo, wo1, bo1, out_o):
    agg = p0[...] + p1[...] + dinvsq2[...] * h1lin[...] + bc1[...]
    h1 = jax.nn.relu(_bn_rows(agg, g1[...], be1[...]))
    oht = (lax.broadcasted_iota(jnp.int32, (G, N), 0) == bids[...]).astype(jnp.float32)
    psum = jnp.dot(oht, h1, preferred_element_type=jnp.float32)
    counts = jnp.sum(oht, axis=1, keepdims=True)
    pooled = psum / jnp.maximum(counts, 1.0)
    sv = jnp.dot(speed[...], wsp[...], preferred_element_type=jnp.float32) + bsp[...]
    venc = jax.nn.relu(_bn_rows(sv, gsp[...], besp[...]))
    w = wrc[...]
    zpad = jnp.zeros((G, 1), jnp.float32)
    rap = jnp.concatenate([zpad, ra[...], zpad], axis=1)
    rbp = jnp.concatenate([zpad, rb[...], zpad], axis=1)
    rc = brc[...]
    for k in range(3):
        rc = rc + rap[:, k:k + 10] * w[0:1, k:k + 1] + rbp[:, k:k + 10] * w[1:2, k:k + 1]
    m = jnp.mean(rc)
    v = jnp.mean((rc - m) * (rc - m))
    rcn = jax.nn.relu(grc[...] * (rc - m) * lax.rsqrt(v + 1e-5) + berc[...])
    renc = jnp.dot(rcn, wrl[...], preferred_element_type=jnp.float32) + brl[...]
    hh = jnp.concatenate([pooled, venc, renc], axis=1)
    s0 = jnp.dot(hh, wo0[...], preferred_element_type=jnp.float32) + bo0[...]
    a = jax.nn.relu(_bn_rows(s0, go[...], beo[...]))
    out_o[...] = jnp.dot(a, wo1[...], preferred_element_type=jnp.float32) + bo1[...]


def _tc3(*args):
    return pl.pallas_call(
        _tc3_body,
        out_shape=jax.ShapeDtypeStruct((G, 8), jnp.float32),
    )(*args)


def kernel(x, edge_weight, speed, route, params, edge_index, batch_ids):
    p = params
    src = edge_index[0]
    dst = edge_index[1]
    pad = NEP - NE
    zi = jnp.zeros((pad,), jnp.int32)
    srcp = jnp.concatenate([src, zi]).reshape(ROWS2D, 128)
    dstp = jnp.concatenate([dst, zi]).reshape(ROWS2D, 128)
    wp = jnp.concatenate([edge_weight, jnp.zeros((pad,), jnp.float32)]).reshape(ROWS2D, 128)

    degp = _sc_deg(dstp, wp).reshape(NW, NPAD)
    h0lin, dinv, dinvsq = _tc1(x, p['W_emb'], p['b_emb'], p['g_emb'], p['be_emb'],
                               p['W_c0'], degp)
    dinvsq2 = dinvsq.reshape(N, 1)
    norm2d, p00, p01 = _sc_layer0(srcp, dstp, wp, dinv, h0lin)
    h1lin = _tc2(p00[:N], p01[:N], h0lin, dinvsq2, p['b_c0'], p['g_bn0'],
                 p['be_bn0'], p['W_c1'])
    p10, p11 = _sc_layer1(srcp, dstp, norm2d, h1lin)
    out = _tc3(p10[:N], p11[:N], h1lin, dinvsq2, p['b_c1'], p['g_bn1'], p['be_bn1'],
               batch_ids.reshape(1, N),
               speed, p['W_sp'], p['b_sp'], p['g_sp'], p['be_sp'],
               route[:, :, 0], route[:, :, 1],
               p['W_rc'].reshape(2, 3), p['b_rc'], p['g_rc'], p['be_rc'],
               p['W_rl'], p['b_rl'],
               p['W_o0'], p['b_o0'], p['g_o'], p['be_o'], p['W_o1'], p['b_o1'])
    return out


# speedup vs baseline: 3.0652x; 1.0002x over previous
import functools

import jax
import jax.numpy as jnp
from jax import lax
from jax.experimental import pallas as pl
from jax.experimental.pallas import tpu as pltpu
from jax.experimental.pallas import tpu_sc as plsc

N = 10000
NE = 320000
F = 128
H = 16
G = 256
NC, NS, L = 2, 16, 16
NW = NC * NS
NEP = 327680
ROWS2D = NEP // 128
EPT = NEP // NW
CH = 2048
GRP = CH // 128
CPW = EPT // CH
RGPW = EPT // 128
NPAD = 10240
NPT = NPAD // NS

_mesh = plsc.VectorSubcoreMesh(core_axis_name="c", subcore_axis_name="s")
_sc_params = pltpu.CompilerParams(needs_layout_passes=False,
                                  use_tc_tiling_on_sc=False)


@functools.partial(
    pl.kernel,
    out_type=jax.ShapeDtypeStruct((NW * NPAD,), jnp.float32),
    mesh=_mesh,
    compiler_params=_sc_params,
    scratch_types=[
        pltpu.VMEM((NPAD,), jnp.float32),
        pltpu.VMEM((GRP, 128), jnp.int32),
        pltpu.VMEM((GRP, 128), jnp.float32),
    ],
)
def _sc_deg(dst_hbm, w_hbm, out_hbm, acc, dstb, wb):
    cid = lax.axis_index("c")
    sid = lax.axis_index("s")
    wid = sid * NC + cid
    zero16 = jnp.zeros((L,), jnp.float32)
    ji = lax.iota(jnp.int32, L)

    def _zero(i, c):
        acc[pl.ds(i * L, L)] = zero16
        return c
    lax.fori_loop(0, NPAD // L, _zero, 0)

    def _chunk(c, carry):
        row0 = wid * RGPW + c * GRP
        pltpu.sync_copy(dst_hbm.at[pl.ds(row0, GRP)], dstb)
        pltpu.sync_copy(w_hbm.at[pl.ds(row0, GRP)], wb)

        def _vreg(gk, c2):
            g = gk // (128 // L)
            j = gk % (128 // L)
            dv = dstb[g, pl.ds(j * L, L)]
            wv = wb[g, pl.ds(j * L, L)]
            for lane in range(L):
                plsc.addupdate_scatter(acc, [dv], wv, mask=ji == lane)
            return c2
        lax.fori_loop(0, CH // L, _vreg, 0)
        return carry
    lax.fori_loop(0, CPW, _chunk, 0)
    pltpu.sync_copy(acc, out_hbm.at[pl.ds(wid * NPAD, NPAD)])


def _make_sc_layer(compute_norm):
    n_out = ([jax.ShapeDtypeStruct((NEP,), jnp.float32)] if compute_norm else []) + [
        jax.ShapeDtypeStruct((NPAD, H), jnp.float32),
        jax.ShapeDtypeStruct((NPAD, H), jnp.float32),
    ]
    scratch = [
        pltpu.VMEM((2, CH), jnp.int32),
        pltpu.VMEM((2, CH), jnp.int32),
        pltpu.VMEM((2, CH), jnp.float32),
        pltpu.VMEM((2, CH, H), jnp.float32),
        pltpu.VMEM((NPT, H), jnp.float32),
        pltpu.VMEM_SHARED((NPAD, H), jnp.float32),
        pltpu.SemaphoreType.DMA,
        pltpu.SemaphoreType.DMA,
    ]
    if compute_norm:
        scratch = [pltpu.VMEM((2, CH), jnp.float32),
                   pltpu.VMEM((N,), jnp.float32),
                   ] + scratch

    def body(*refs):
        if compute_norm:
            (src_hbm, dst_hbm, w_hbm, dinv_hbm, h_hbm,
             norm_hbm, p0_hbm, p1_hbm,
             wb, dinv_v, srcb, dstb, normb, rows, zbuf, spacc, sem0, sem1) = refs
        else:
            (src_hbm, dst_hbm, norm_hbm, h_hbm,
             p0_hbm, p1_hbm,
             srcb, dstb, normb, rows, zbuf, spacc, sem0, sem1) = refs
        sems = (sem0, sem1)
        cid = lax.axis_index("c")
        sid = lax.axis_index("s")
        wid = sid * NC + cid
        zero16 = jnp.zeros((H,), jnp.float32)

        def _zrow(i, c):
            zbuf[i, :] = zero16
            return c
        lax.fori_loop(0, NPT, _zrow, 0)
        pltpu.sync_copy(zbuf, spacc.at[pl.ds(sid * NPT, NPT)])
        if compute_norm:
            pltpu.sync_copy(dinv_hbm, dinv_v)
        plsc.subcore_barrier()

        def _load_idx(c, b):
            e0 = wid * EPT + c * CH
            pltpu.sync_copy(src_hbm.at[pl.ds(e0, CH)], srcb.at[b])
            pltpu.sync_copy(dst_hbm.at[pl.ds(e0, CH)], dstb.at[b])
            if compute_norm:
                pltpu.sync_copy(w_hbm.at[pl.ds(e0, CH)], wb.at[b])
            else:
                pltpu.sync_copy(norm_hbm.at[pl.ds(e0, CH)], normb.at[b])

        def _fire_gather(b):
            return [pltpu.async_copy(h_hbm.at[srcb.at[b]], rows.at[b], sems[b])]

        _load_idx(0, 0)
        pend = _fire_gather(0)
        for c in range(CPW):
            b = c % 2
            nb = (c + 1) % 2
            if c + 1 < CPW:
                _load_idx(c + 1, nb)
                nxt = _fire_gather(nb)
            for d in pend:
                d.wait()
            if c + 1 < CPW:
                pend = nxt

            def _vreg(k, c2):
                if compute_norm:
                    sv = srcb[b, pl.ds(k * L, L)]
                    dv = dstb[b, pl.ds(k * L, L)]
                    wv = wb[b, pl.ds(k * L, L)]
                    nv = (plsc.load_gather(dinv_v, [sv]) * wv
                          * plsc.load_gather(dinv_v, [dv]))
                    normb[b, pl.ds(k * L, L)] = nv
                else:
                    nv = normb[b, pl.ds(k * L, L)]
                for lane in range(L):
                    e = k * L + lane
                    rows[b, e, :] = rows[b, e, :] * nv[lane]
                return c2
            lax.fori_loop(0, CH // L, _vreg, 0)
            pltpu.sync_copy(rows.at[b], spacc.at[dstb.at[b]], add=True)
            if compute_norm:
                e0 = wid * EPT + c * CH
                pltpu.sync_copy(normb.at[b], norm_hbm.at[pl.ds(e0, CH)])
        plsc.subcore_barrier()

        @pl.when(cid == 0)
        def _():
            pltpu.sync_copy(spacc.at[pl.ds(sid * NPT, NPT)],
                            p0_hbm.at[pl.ds(sid * NPT, NPT)])

        @pl.when(cid == 1)
        def _():
            pltpu.sync_copy(spacc.at[pl.ds(sid * NPT, NPT)],
                            p1_hbm.at[pl.ds(sid * NPT, NPT)])

    return functools.partial(pl.kernel, out_type=n_out, mesh=_mesh,
                             compiler_params=_sc_params,
                             scratch_types=scratch)(body)


_sc_layer0 = _make_sc_layer(True)
_sc_layer1 = _make_sc_layer(False)


def _bn_rows(s, g, b):
    m = jnp.mean(s, axis=0, keepdims=True)
    v = jnp.mean((s - m) * (s - m), axis=0, keepdims=True)
    return g * (s - m) * lax.rsqrt(v + 1e-5) + b


def _tc1_body(x, wemb, bemb, gemb, beemb, wc0, degp,
              h0lin_o, dinv_o, dinvsq_o):
    s = jnp.dot(x[...], wemb[...], preferred_element_type=jnp.float32) + bemb[...]
    h = jax.nn.relu(_bn_rows(s, gemb[...], beemb[...]))
    h0lin_o[...] = jnp.dot(h, wc0[...], preferred_element_type=jnp.float32)
    deg = jnp.sum(degp[...], axis=0)[:N] + 1.0
    dinv_o[...] = lax.rsqrt(deg)
    dinvsq_o[...] = 1.0 / deg


def _tc1(x, wemb, bemb, gemb, beemb, wc0, degp):
    return pl.pallas_call(
        _tc1_body,
        out_shape=[
            jax.ShapeDtypeStruct((N, H), jnp.float32),
            jax.ShapeDtypeStruct((N,), jnp.float32),
            jax.ShapeDtypeStruct((N,), jnp.float32),
        ],
    )(x, wemb, bemb, gemb, beemb, wc0, degp)


def _tc2_body(p0, p1, h0lin, dinvsq2, bc0, g0, be0, wc1, h1lin_o):
    agg = p0[...] + p1[...] + dinvsq2[...] * h0lin[...] + bc0[...]
    h0 = jax.nn.relu(_bn_rows(agg, g0[...], be0[...]))
    h1lin_o[...] = jnp.dot(h0, wc1[...], preferred_element_type=jnp.float32)


def _tc2(p0, p1, h0lin, dinvsq2, bc0, g0, be0, wc1):
    return pl.pallas_call(
        _tc2_body,
        out_shape=jax.ShapeDtypeStruct((N, H), jnp.float32),
    )(p0, p1, h0lin, dinvsq2, bc0, g0, be0, wc1)


def _tc3_body(p0, p1, h1lin, dinvsq2, bc1, g1, be1, bids,
              speed, wsp, bsp, gsp, besp,
              ra, rb, wrc, brc, grc, berc, wrl, brl,
              wo0, bo0, go, beo, wo1, bo1, out_o):
    agg = p0[...] + p1[...] + dinvsq2[...] * h1lin[...] + bc1[...]
    h1 = jax.nn.relu(_bn_rows(agg, g1[...], be1[...]))
    oht = (lax.broadcasted_iota(jnp.int32, (G, N), 0) == bids[...]).astype(jnp.float32)
    psum = jnp.dot(oht, h1, preferred_element_type=jnp.float32)
    counts = jnp.sum(oht, axis=1, keepdims=True)
    pooled = psum / jnp.maximum(counts, 1.0)
    sv = jnp.dot(speed[...], wsp[...], preferred_element_type=jnp.float32) + bsp[...]
    venc = jax.nn.relu(_bn_rows(sv, gsp[...], besp[...]))
    w = wrc[...]
    zpad = jnp.zeros((G, 1), jnp.float32)
    rap = jnp.concatenate([zpad, ra[...], zpad], axis=1)
    rbp = jnp.concatenate([zpad, rb[...], zpad], axis=1)
    rc = brc[...]
    for k in range(3):
        rc = rc + rap[:, k:k + 10] * w[0:1, k:k + 1] + rbp[:, k:k + 10] * w[1:2, k:k + 1]
    m = jnp.mean(rc)
    v = jnp.mean((rc - m) * (rc - m))
    rcn = jax.nn.relu(grc[...] * (rc - m) * lax.rsqrt(v + 1e-5) + berc[...])
    renc = jnp.dot(rcn, wrl[...], preferred_element_type=jnp.float32) + brl[...]
    hh = jnp.concatenate([pooled, venc, renc], axis=1)
    s0 = jnp.dot(hh, wo0[...], preferred_element_type=jnp.float32) + bo0[...]
    a = jax.nn.relu(_bn_rows(s0, go[...], beo[...]))
    out_o[...] = jnp.dot(a, wo1[...], preferred_element_type=jnp.float32) + bo1[...]


def _tc3(*args):
    return pl.pallas_call(
        _tc3_body,
        out_shape=jax.ShapeDtypeStruct((G, 8), jnp.float32),
    )(*args)


def kernel(x, edge_weight, speed, route, params, edge_index, batch_ids):
    p = params
    src = edge_index[0]
    dst = edge_index[1]
    pad = NEP - NE
    zi = jnp.zeros((pad,), jnp.int32)
    srcp = jnp.concatenate([src, zi])
    dstp = jnp.concatenate([dst, zi])
    wp = jnp.concatenate([edge_weight, jnp.zeros((pad,), jnp.float32)])
    dstp2 = dstp.reshape(ROWS2D, 128)
    wp2 = wp.reshape(ROWS2D, 128)

    degp = _sc_deg(dstp2, wp2).reshape(NW, NPAD)
    h0lin, dinv, dinvsq = _tc1(x, p['W_emb'], p['b_emb'], p['g_emb'], p['be_emb'],
                               p['W_c0'], degp)
    dinvsq2 = dinvsq.reshape(N, 1)
    norm2d, p00, p01 = _sc_layer0(srcp, dstp, wp, dinv, h0lin)
    h1lin = _tc2(p00[:N], p01[:N], h0lin, dinvsq2, p['b_c0'], p['g_bn0'],
                 p['be_bn0'], p['W_c1'])
    p10, p11 = _sc_layer1(srcp, dstp, norm2d, h1lin)
    out = _tc3(p10[:N], p11[:N], h1lin, dinvsq2, p['b_c1'], p['g_bn1'], p['be_bn1'],
               batch_ids.reshape(1, N),
               speed, p['W_sp'], p['b_sp'], p['g_sp'], p['be_sp'],
               route[:, :, 0], route[:, :, 1],
               p['W_rc'].reshape(2, 3), p['b_rc'], p['g_rc'], p['be_rc'],
               p['W_rl'], p['b_rl'],
               p['W_o0'], p['b_o0'], p['g_o'], p['be_o'], p['W_o1'], p['b_o1'])
    return out


# speedup vs baseline: 3.1507x; 1.0279x over previous
import functools

import jax
import jax.numpy as jnp
from jax import lax
from jax.experimental import pallas as pl
from jax.experimental.pallas import tpu as pltpu
from jax.experimental.pallas import tpu_sc as plsc

N = 10000
NE = 320000
F = 128
H = 16
G = 256
NC, NS, L = 2, 16, 16
NW = NC * NS
NEP = 327680
ROWS2D = NEP // 128
EPT = NEP // NW
CH = 2048
GRP = CH // 128
CPW = EPT // CH
RGPW = EPT // 128
NPAD = 10240
NPT = NPAD // NS

_mesh = plsc.VectorSubcoreMesh(core_axis_name="c", subcore_axis_name="s")
_sc_params = pltpu.CompilerParams(needs_layout_passes=False,
                                  use_tc_tiling_on_sc=False)


@functools.partial(
    pl.kernel,
    out_type=jax.ShapeDtypeStruct((NW, NPAD), jnp.float32),
    mesh=_mesh,
    compiler_params=_sc_params,
    scratch_types=[
        pltpu.VMEM((NPAD,), jnp.float32),
        pltpu.VMEM((GRP, 128), jnp.int32),
        pltpu.VMEM((GRP, 128), jnp.float32),
    ],
)
def _sc_deg(dst_hbm, w_hbm, out_hbm, acc, dstb, wb):
    cid = lax.axis_index("c")
    sid = lax.axis_index("s")
    wid = sid * NC + cid
    zero16 = jnp.zeros((L,), jnp.float32)
    ji = lax.iota(jnp.int32, L)

    def _zero(i, c):
        acc[pl.ds(i * L, L)] = zero16
        return c
    lax.fori_loop(0, NPAD // L, _zero, 0)

    def _chunk(c, carry):
        row0 = wid * RGPW + c * GRP
        pltpu.sync_copy(dst_hbm.at[pl.ds(row0, GRP)], dstb)
        pltpu.sync_copy(w_hbm.at[pl.ds(row0, GRP)], wb)

        def _vreg(gk, c2):
            g = gk // (128 // L)
            j = gk % (128 // L)
            dv = dstb[g, pl.ds(j * L, L)]
            wv = wb[g, pl.ds(j * L, L)]
            for lane in range(L):
                plsc.addupdate_scatter(acc, [dv], wv, mask=ji == lane)
            return c2
        lax.fori_loop(0, CH // L, _vreg, 0)
        return carry
    lax.fori_loop(0, CPW, _chunk, 0)
    pltpu.sync_copy(acc, out_hbm.at[wid])


def _make_sc_layer(compute_norm):
    n_out = ([jax.ShapeDtypeStruct((NEP,), jnp.float32)] if compute_norm else []) + [
        jax.ShapeDtypeStruct((NPAD, H), jnp.float32),
        jax.ShapeDtypeStruct((NPAD, H), jnp.float32),
    ]
    scratch = [
        pltpu.VMEM((2, CH), jnp.int32),
        pltpu.VMEM((2, CH), jnp.int32),
        pltpu.VMEM((2, CH), jnp.float32),
        pltpu.VMEM((2, CH, H), jnp.float32),
        pltpu.VMEM((NPT, H), jnp.float32),
        pltpu.VMEM_SHARED((NPAD, H), jnp.float32),
        pltpu.SemaphoreType.DMA,
        pltpu.SemaphoreType.DMA,
    ]
    if compute_norm:
        scratch = [pltpu.VMEM((2, CH), jnp.float32),
                   pltpu.VMEM((N,), jnp.float32),
                   ] + scratch

    def body(*refs):
        if compute_norm:
            (src_hbm, dst_hbm, w_hbm, dinv_hbm, h_hbm,
             norm_hbm, p0_hbm, p1_hbm,
             wb, dinv_v, srcb, dstb, normb, rows, zbuf, spacc, sem0, sem1) = refs
        else:
            (src_hbm, dst_hbm, norm_hbm, h_hbm,
             p0_hbm, p1_hbm,
             srcb, dstb, normb, rows, zbuf, spacc, sem0, sem1) = refs
        sems = (sem0, sem1)
        cid = lax.axis_index("c")
        sid = lax.axis_index("s")
        wid = sid * NC + cid
        zero16 = jnp.zeros((H,), jnp.float32)

        def _zrow(i, c):
            zbuf[i, :] = zero16
            return c
        lax.fori_loop(0, NPT, _zrow, 0)
        pltpu.sync_copy(zbuf, spacc.at[pl.ds(sid * NPT, NPT)])
        if compute_norm:
            pltpu.sync_copy(dinv_hbm, dinv_v)
        plsc.subcore_barrier()

        def _load_idx(c, b):
            e0 = wid * EPT + c * CH
            pltpu.sync_copy(src_hbm.at[pl.ds(e0, CH)], srcb.at[b])
            pltpu.sync_copy(dst_hbm.at[pl.ds(e0, CH)], dstb.at[b])
            if compute_norm:
                pltpu.sync_copy(w_hbm.at[pl.ds(e0, CH)], wb.at[b])
            else:
                pltpu.sync_copy(norm_hbm.at[pl.ds(e0, CH)], normb.at[b])

        def _fire_gather(b):
            return [pltpu.async_copy(h_hbm.at[srcb.at[b]], rows.at[b], sems[b])]

        _load_idx(0, 0)
        pend = _fire_gather(0)
        for c in range(CPW):
            b = c % 2
            nb = (c + 1) % 2
            if c + 1 < CPW:
                _load_idx(c + 1, nb)
                nxt = _fire_gather(nb)
            for d in pend:
                d.wait()
            if c + 1 < CPW:
                pend = nxt

            def _vreg(k, c2):
                if compute_norm:
                    sv = srcb[b, pl.ds(k * L, L)]
                    dv = dstb[b, pl.ds(k * L, L)]
                    wv = wb[b, pl.ds(k * L, L)]
                    nv = (plsc.load_gather(dinv_v, [sv]) * wv
                          * plsc.load_gather(dinv_v, [dv]))
                    normb[b, pl.ds(k * L, L)] = nv
                else:
                    nv = normb[b, pl.ds(k * L, L)]
                for lane in range(L):
                    e = k * L + lane
                    rows[b, e, :] = rows[b, e, :] * nv[lane]
                return c2
            lax.fori_loop(0, CH // L, _vreg, 0)
            pltpu.sync_copy(rows.at[b], spacc.at[dstb.at[b]], add=True)
            if compute_norm:
                e0 = wid * EPT + c * CH
                pltpu.sync_copy(normb.at[b], norm_hbm.at[pl.ds(e0, CH)])
        plsc.subcore_barrier()

        @pl.when(cid == 0)
        def _():
            pltpu.sync_copy(spacc.at[pl.ds(sid * NPT, NPT)],
                            p0_hbm.at[pl.ds(sid * NPT, NPT)])

        @pl.when(cid == 1)
        def _():
            pltpu.sync_copy(spacc.at[pl.ds(sid * NPT, NPT)],
                            p1_hbm.at[pl.ds(sid * NPT, NPT)])

    return functools.partial(pl.kernel, out_type=n_out, mesh=_mesh,
                             compiler_params=_sc_params,
                             scratch_types=scratch)(body)


_sc_layer0 = _make_sc_layer(True)
_sc_layer1 = _make_sc_layer(False)


def _bn_rows(s, g, b):
    m = jnp.mean(s, axis=0, keepdims=True)
    v = jnp.mean((s - m) * (s - m), axis=0, keepdims=True)
    return g * (s - m) * lax.rsqrt(v + 1e-5) + b


def _tc1_body(x, wemb, bemb, gemb, beemb, wc0, degp,
              h0lin_o, dinv_o, dinvsq_o):
    s = jnp.dot(x[...], wemb[...], preferred_element_type=jnp.float32) + bemb[...]
    h = jax.nn.relu(_bn_rows(s, gemb[...], beemb[...]))
    h0lin_o[...] = jnp.dot(h, wc0[...], preferred_element_type=jnp.float32)
    deg = jnp.sum(degp[...], axis=0)[:N] + 1.0
    dinv_o[...] = lax.rsqrt(deg)
    dinvsq_o[...] = (1.0 / deg)[:, None]


def _tc1(x, wemb, bemb, gemb, beemb, wc0, degp):
    return pl.pallas_call(
        _tc1_body,
        out_shape=[
            jax.ShapeDtypeStruct((N, H), jnp.float32),
            jax.ShapeDtypeStruct((N,), jnp.float32),
            jax.ShapeDtypeStruct((N, 1), jnp.float32),
        ],
    )(x, wemb, bemb, gemb, beemb, wc0, degp)


def _tc2_body(p0, p1, h0lin, dinvsq2, bc0, g0, be0, wc1, h1lin_o):
    agg = p0[...][:N] + p1[...][:N] + dinvsq2[...] * h0lin[...] + bc0[...]
    h0 = jax.nn.relu(_bn_rows(agg, g0[...], be0[...]))
    h1lin_o[...] = jnp.dot(h0, wc1[...], preferred_element_type=jnp.float32)


def _tc2(p0, p1, h0lin, dinvsq2, bc0, g0, be0, wc1):
    return pl.pallas_call(
        _tc2_body,
        out_shape=jax.ShapeDtypeStruct((N, H), jnp.float32),
    )(p0, p1, h0lin, dinvsq2, bc0, g0, be0, wc1)


def _tc3_body(p0, p1, h1lin, dinvsq2, bc1, g1, be1, bids,
              speed, wsp, bsp, gsp, besp,
              ra, rb, wrc, brc, grc, berc, wrl, brl,
              wo0, bo0, go, beo, wo1, bo1, out_o):
    agg = p0[...][:N] + p1[...][:N] + dinvsq2[...] * h1lin[...] + bc1[...]
    h1 = jax.nn.relu(_bn_rows(agg, g1[...], be1[...]))
    oht = (lax.broadcasted_iota(jnp.int32, (G, N), 0) == bids[...]).astype(jnp.float32)
    psum = jnp.dot(oht, h1, preferred_element_type=jnp.float32)
    counts = jnp.sum(oht, axis=1, keepdims=True)
    pooled = psum / jnp.maximum(counts, 1.0)
    sv = jnp.dot(speed[...], wsp[...], preferred_element_type=jnp.float32) + bsp[...]
    venc = jax.nn.relu(_bn_rows(sv, gsp[...], besp[...]))
    w = wrc[...]
    zpad = jnp.zeros((G, 1), jnp.float32)
    rap = jnp.concatenate([zpad, ra[...], zpad], axis=1)
    rbp = jnp.concatenate([zpad, rb[...], zpad], axis=1)
    rc = brc[...]
    for k in range(3):
        rc = rc + rap[:, k:k + 10] * w[0:1, k:k + 1] + rbp[:, k:k + 10] * w[1:2, k:k + 1]
    m = jnp.mean(rc)
    v = jnp.mean((rc - m) * (rc - m))
    rcn = jax.nn.relu(grc[...] * (rc - m) * lax.rsqrt(v + 1e-5) + berc[...])
    renc = jnp.dot(rcn, wrl[...], preferred_element_type=jnp.float32) + brl[...]
    hh = jnp.concatenate([pooled, venc, renc], axis=1)
    s0 = jnp.dot(hh, wo0[...], preferred_element_type=jnp.float32) + bo0[...]
    a = jax.nn.relu(_bn_rows(s0, go[...], beo[...]))
    out_o[...] = jnp.dot(a, wo1[...], preferred_element_type=jnp.float32) + bo1[...]


def _tc3(*args):
    return pl.pallas_call(
        _tc3_body,
        out_shape=jax.ShapeDtypeStruct((G, 8), jnp.float32),
    )(*args)


def kernel(x, edge_weight, speed, route, params, edge_index, batch_ids):
    p = params
    src = edge_index[0]
    dst = edge_index[1]
    pad = NEP - NE
    zi = jnp.zeros((pad,), jnp.int32)
    srcp = jnp.concatenate([src, zi])
    dstp = jnp.concatenate([dst, zi])
    wp = jnp.concatenate([edge_weight, jnp.zeros((pad,), jnp.float32)])
    dstp2 = dstp.reshape(ROWS2D, 128)
    wp2 = wp.reshape(ROWS2D, 128)

    degp = _sc_deg(dstp2, wp2)
    h0lin, dinv, dinvsq = _tc1(x, p['W_emb'], p['b_emb'], p['g_emb'], p['be_emb'],
                               p['W_c0'], degp)
    dinvsq2 = dinvsq
    norm2d, p00, p01 = _sc_layer0(srcp, dstp, wp, dinv, h0lin)
    h1lin = _tc2(p00, p01, h0lin, dinvsq2, p['b_c0'], p['g_bn0'],
                 p['be_bn0'], p['W_c1'])
    p10, p11 = _sc_layer1(srcp, dstp, norm2d, h1lin)
    out = _tc3(p10, p11, h1lin, dinvsq2, p['b_c1'], p['g_bn1'], p['be_bn1'],
               batch_ids,
               speed, p['W_sp'], p['b_sp'], p['g_sp'], p['be_sp'],
               route[:, :, 0], route[:, :, 1],
               p['W_rc'].reshape(2, 3), p['b_rc'], p['g_rc'], p['be_rc'],
               p['W_rl'], p['b_rl'],
               p['W_o0'], p['b_o0'], p['g_o'], p['be_o'], p['W_o1'], p['b_o1'])
    return out
